# baseline (speedup 1.0000x reference)
import jax
import jax.numpy as jnp
from jax.experimental import pallas as pl
from jax.experimental.pallas import tpu as pltpu

N = 4096
S = 64
C_IN = 6
RADII = (0.1, 0.4, 0.8, 1.6)
NSAMPLES = (4096, 1024, 256, 64)
JT = 128
NJT = N // JT


def _fps_kernel(px_ref, py_ref, pz_ref, nx_ref, ny_ref, nz_ref):
    px = px_ref[...]
    py = py_ref[...]
    pz = pz_ref[...]
    lin = (jax.lax.broadcasted_iota(jnp.int32, (32, 128), 0) * 128
           + jax.lax.broadcasted_iota(jnp.int32, (32, 128), 1)
           ).astype(jnp.float32)
    iota_s = jax.lax.broadcasted_iota(jnp.int32, (1, S), 1).astype(jnp.float32)

    def body(i, st):
        dist, far, nx, ny, nz = st
        sel = (lin == far).astype(jnp.float32)
        cx = jnp.sum(sel * px)
        cy = jnp.sum(sel * py)
        cz = jnp.sum(sel * pz)
        fi = i.astype(jnp.float32)
        nx = jnp.where(iota_s == fi, cx, nx)
        ny = jnp.where(iota_s == fi, cy, ny)
        nz = jnp.where(iota_s == fi, cz, nz)
        d = (px - cx) ** 2 + (py - cy) ** 2 + (pz - cz) ** 2
        dist = jnp.minimum(dist, d)
        m = jnp.max(dist)
        far = jnp.min(jnp.where(dist == m, lin, jnp.float32(1e9)))
        return dist, far, nx, ny, nz

    st0 = (jnp.full((32, 128), 1e10, jnp.float32), jnp.float32(0.0),
           jnp.zeros((1, S), jnp.float32), jnp.zeros((1, S), jnp.float32),
           jnp.zeros((1, S), jnp.float32))
    _, _, nx, ny, nz = jax.lax.fori_loop(0, S, body, st0)
    nx_ref[...] = nx
    ny_ref[...] = ny
    nz_ref[...] = nz


def _sa_kernel(pc2d_ref, pcT_ref, featsT_ref, nxyz_ref, *rest):
    prm = rest[:52]
    out_ref = rest[52]
    p_scr = rest[53]
    w_scr = rest[54]

    pc2d = pc2d_ref[...]
    pcT = pcT_ref[...]
    featsT = featsT_ref[...]
    nxyz = nxyz_ref[...]

    nn = jnp.sum(nxyz * nxyz, axis=1, keepdims=True)
    pp = jnp.sum(pcT * pcT, axis=0, keepdims=True)
    cross = jnp.dot(nxyz, pcT, preferred_element_type=jnp.float32)
    d = nn + pp - 2.0 * cross

    for b in range(4):
        (w0f_t, w0x_t, b0, g0, be0,
         w1t, b1, g1, be1,
         w2t, b2, g2, be2) = [r[...] for r in prm[13 * b:13 * (b + 1)]]
        ns = float(NSAMPLES[b])
        m_count = float(S) * ns
        r2 = RADII[b] * RADII[b]

        mask = (d <= r2).astype(jnp.float32)
        c = mask
        sh = 1
        while sh < N:
            c = c + jnp.concatenate(
                [jnp.zeros((S, sh), jnp.float32), c[:, :N - sh]], axis=1)
            sh *= 2
        rank_excl = c - mask
        k_in = jnp.sum(mask, axis=1, keepdims=True)
        k_cl = jnp.minimum(k_in, ns)
        sel = mask * (rank_excl < ns).astype(jnp.float32)
        first = mask * (rank_excl == 0.0).astype(jnp.float32)
        w = sel + first * (ns - k_cl)
        w_scr[...] = w

        p = (jnp.dot(featsT, w0f_t, preferred_element_type=jnp.float32)
             + jnp.dot(pc2d, w0x_t, preferred_element_type=jnp.float32)
             + b0)
        p_scr[...] = p
        q = jnp.dot(nxyz, w0x_t, preferred_element_type=jnp.float32)

        cw = jnp.sum(w, axis=0, keepdims=True)
        sq_sum = jnp.sum(q, axis=0, keepdims=True)
        s1 = jnp.dot(cw, p, preferred_element_type=jnp.float32) - ns * sq_sum
        t_sw = jnp.dot(w, p, preferred_element_type=jnp.float32)
        s1sq = (jnp.dot(cw, p * p, preferred_element_type=jnp.float32)
                - 2.0 * jnp.sum(q * t_sw, axis=0, keepdims=True)
                + ns * jnp.sum(q * q, axis=0, keepdims=True))
        mu1 = s1 / m_count
        var1 = s1sq / m_count - mu1 * mu1
        sc1 = g0 / jnp.sqrt(var1 + 1e-5)
        of1 = be0 - mu1 * sc1

        def h1_tile(t):
            pt = p_scr[pl.ds(t * JT, JT), :]
            y1 = pt[None, :, :] - q[:, None, :]
            return jnp.maximum(y1 * sc1 + of1, 0.0)

        def w_tile(t):
            return w_scr[:, pl.ds(t * JT, JT)][:, :, None]

        def body_b(t, acc):
            h1 = h1_tile(t).reshape(S * JT, 128)
            y2 = (jnp.dot(h1, w1t, preferred_element_type=jnp.float32) + b1
                  ).reshape(S, JT, 128)
            wt = w_tile(t)
            sa, sb = acc
            return (sa + jnp.sum(y2 * wt, axis=(0, 1), keepdims=True),
                    sb + jnp.sum(y2 * y2 * wt, axis=(0, 1), keepdims=True))

        z128 = jnp.zeros((1, 1, 128), jnp.float32)
        s2, s2sq = jax.lax.fori_loop(0, NJT, body_b, (z128, z128))
        mu2 = s2[0] / m_count
        var2 = s2sq[0] / m_count - mu2 * mu2
        sc2 = g1 / jnp.sqrt(var2 + 1e-5)
        of2 = be1 - mu2 * sc2

        def h2_tile(t):
            h1 = h1_tile(t).reshape(S * JT, 128)
            y2 = jnp.dot(h1, w1t, preferred_element_type=jnp.float32) + b1
            return jnp.maximum(y2 * sc2 + of2, 0.0)

        def body_c(t, acc):
            h2 = h2_tile(t)
            y3 = (jnp.dot(h2, w2t, preferred_element_type=jnp.float32) + b2
                  ).reshape(S, JT, 256)
            wt = w_tile(t)
            sa, sb = acc
            return (sa + jnp.sum(y3 * wt, axis=(0, 1), keepdims=True),
                    sb + jnp.sum(y3 * y3 * wt, axis=(0, 1), keepdims=True))

        z256 = jnp.zeros((1, 1, 256), jnp.float32)
        s3, s3sq = jax.lax.fori_loop(0, NJT, body_c, (z256, z256))
        mu3 = s3[0] / m_count
        var3 = s3sq[0] / m_count - mu3 * mu3
        sc3 = g2 / jnp.sqrt(var3 + 1e-5)
        of3 = be2 - mu3 * sc3

        def body_d(t, acc):
            h2 = h2_tile(t)
            y3 = (jnp.dot(h2, w2t, preferred_element_type=jnp.float32) + b2
                  ).reshape(S, JT, 256)
            h3 = jnp.maximum(y3 * sc3 + of3, 0.0)
            selm = w_tile(t) > 0.0
            hm = jnp.where(selm, h3, jnp.float32(-1e30))
            return jnp.maximum(acc, jnp.max(hm, axis=1))

        mx0 = jnp.full((S, 256), -1e30, jnp.float32)
        mx = jax.lax.fori_loop(0, NJT, body_d, mx0)
        out_ref[:, 256 * b:256 * (b + 1)] = mx


def _fp_kernel(pc2d_ref, nxyzT_ref, featsT_ref, l1_ref,
               w0i_t_ref, w0f_t_ref, fb0_ref, fg0_ref, fbe0_ref,
               w1t_ref, fb1_ref, fg1_ref, fbe1_ref,
               c1t_ref, c1b_ref, bn1g_ref, bn1be_ref,
               c2w_ref, c2b_ref, out_ref):
    pc2d = pc2d_ref[...]
    nxyzT = nxyzT_ref[...]
    featsT = featsT_ref[...]
    l1 = l1_ref[...]

    pp = jnp.sum(pc2d * pc2d, axis=1, keepdims=True)
    nn = jnp.sum(nxyzT * nxyzT, axis=0, keepdims=True)
    cross = jnp.dot(pc2d, nxyzT, preferred_element_type=jnp.float32)
    d = pp + nn - 2.0 * cross

    colidx = jax.lax.broadcasted_iota(jnp.int32, (N, S), 1).astype(jnp.float32)
    cur = d
    ohs, ws = [], []
    for _ in range(3):
        m = jnp.min(cur, axis=1, keepdims=True)
        cand = jnp.where(cur == m, colidx, jnp.float32(1e9))
        mi = jnp.min(cand, axis=1, keepdims=True)
        oh = (colidx == mi).astype(jnp.float32)
        ohs.append(oh)
        ws.append(1.0 / jnp.maximum(m, 1e-10))
        cur = jnp.where(oh > 0.0, jnp.float32(1e30), cur)
    wsum = ws[0] + ws[1] + ws[2]
    mat = (ohs[0] * (ws[0] / wsum) + ohs[1] * (ws[1] / wsum)
           + ohs[2] * (ws[2] / wsum))

    interp = jnp.dot(mat, l1, preferred_element_type=jnp.float32)

    x = (jnp.dot(interp, w0i_t_ref[...], preferred_element_type=jnp.float32)
         + jnp.dot(featsT, w0f_t_ref[...], preferred_element_type=jnp.float32)
         + fb0_ref[...])
    mu = jnp.mean(x, axis=0, keepdims=True)
    var = jnp.mean(x * x, axis=0, keepdims=True) - mu * mu
    x = jnp.maximum((x - mu) / jnp.sqrt(var + 1e-5) * fg0_ref[...]
                    + fbe0_ref[...], 0.0)

    x = jnp.dot(x, w1t_ref[...], preferred_element_type=jnp.float32) + fb1_ref[...]
    mu = jnp.mean(x, axis=0, keepdims=True)
    var = jnp.mean(x * x, axis=0, keepdims=True) - mu * mu
    x = jnp.maximum((x - mu) / jnp.sqrt(var + 1e-5) * fg1_ref[...]
                    + fbe1_ref[...], 0.0)

    x = jnp.dot(x, c1t_ref[...], preferred_element_type=jnp.float32) + c1b_ref[...]
    mu = jnp.mean(x, axis=0, keepdims=True)
    var = jnp.mean(x * x, axis=0, keepdims=True) - mu * mu
    x = (x - mu) / jnp.sqrt(var + 1e-5) * bn1g_ref[...] + bn1be_ref[...]
    x = jnp.where(x > 0.0, x, 0.01 * x)

    t = jnp.sum(x * c2w_ref[...], axis=1, keepdims=True) + c2b_ref[...]
    tm = jnp.max(t, axis=1, keepdims=True)
    shf = t - tm
    lse = jnp.log(jnp.sum(jnp.exp(shf), axis=1, keepdims=True))
    ls = shf - lse
    out_ref[...] = 1.0 / (1.0 + jnp.exp(-ls))


def kernel(pc, features, params):
    f32 = jnp.float32
    pc2d = pc[0]
    featsT = jnp.transpose(features[0])
    px = pc2d[:, 0].reshape(32, 128)
    py = pc2d[:, 1].reshape(32, 128)
    pz = pc2d[:, 2].reshape(32, 128)

    nx, ny, nz = pl.pallas_call(
        _fps_kernel,
        out_shape=[jax.ShapeDtypeStruct((1, S), f32)] * 3,
    )(px, py, pz)
    nxyz = jnp.concatenate(
        [nx.reshape(S, 1), ny.reshape(S, 1), nz.reshape(S, 1)], axis=1)

    sa_args = [pc2d, jnp.transpose(pc2d), featsT, nxyz]
    for b in range(4):
        w0 = params['sa%d_W0' % b]
        sa_args += [
            jnp.transpose(w0[:, :C_IN]), jnp.transpose(w0[:, C_IN:]),
            params['sa%d_b0' % b].reshape(1, 128),
            params['sa%d_g0' % b].reshape(1, 128),
            params['sa%d_be0' % b].reshape(1, 128),
            jnp.transpose(params['sa%d_W1' % b]),
            params['sa%d_b1' % b].reshape(1, 128),
            params['sa%d_g1' % b].reshape(1, 128),
            params['sa%d_be1' % b].reshape(1, 128),
            jnp.transpose(params['sa%d_W2' % b]),
            params['sa%d_b2' % b].reshape(1, 256),
            params['sa%d_g2' % b].reshape(1, 256),
            params['sa%d_be2' % b].reshape(1, 256),
        ]
    l1 = pl.pallas_call(
        _sa_kernel,
        out_shape=jax.ShapeDtypeStruct((S, 1024), f32),
        scratch_shapes=[pltpu.VMEM((N, 128), f32), pltpu.VMEM((S, N), f32)],
    )(*sa_args)

    fw0 = params['fp_W0']
    fp_args = [
        pc2d, jnp.transpose(nxyz), featsT, l1,
        jnp.transpose(fw0[:, :1024]), jnp.transpose(fw0[:, 1024:]),
        params['fp_b0'].reshape(1, 128),
        params['fp_g0'].reshape(1, 128),
        params['fp_be0'].reshape(1, 128),
        jnp.transpose(params['fp_W1']),
        params['fp_b1'].reshape(1, 64),
        params['fp_g1'].reshape(1, 64),
        params['fp_be1'].reshape(1, 64),
        jnp.transpose(params['c1_W']),
        params['c1_b'].reshape(1, 64),
        params['bn1_g'].reshape(1, 64),
        params['bn1_be'].reshape(1, 64),
        params['c2_W'].reshape(1, 64),
        params['c2_b'].reshape(1, 1),
    ]
    out = pl.pallas_call(
        _fp_kernel,
        out_shape=jax.ShapeDtypeStruct((N, 1), f32),
    )(*fp_args)
    return out.reshape(1, 1, N)


# speedup vs baseline: 1.2888x; 1.2888x over previous
import jax
import jax.numpy as jnp
from jax.experimental import pallas as pl
from jax.experimental.pallas import tpu as pltpu

N = 4096
S = 64
C_IN = 6
RADII = (0.1, 0.4, 0.8, 1.6)
NSAMPLES = (4096, 1024, 256, 64)
JT = 128
NJT = N // JT


def _fps_kernel(px_ref, py_ref, pz_ref, nx_ref, ny_ref, nz_ref):
    px = px_ref[...]
    py = py_ref[...]
    pz = pz_ref[...]
    lin = (jax.lax.broadcasted_iota(jnp.int32, (32, 128), 0) * 128
           + jax.lax.broadcasted_iota(jnp.int32, (32, 128), 1)
           ).astype(jnp.float32)
    iota_s = jax.lax.broadcasted_iota(jnp.int32, (1, S), 1).astype(jnp.float32)

    def body(i, st):
        dist, far, nx, ny, nz = st
        sel = (lin == far).astype(jnp.float32)
        cx = jnp.sum(sel * px)
        cy = jnp.sum(sel * py)
        cz = jnp.sum(sel * pz)
        fi = i.astype(jnp.float32)
        nx = jnp.where(iota_s == fi, cx, nx)
        ny = jnp.where(iota_s == fi, cy, ny)
        nz = jnp.where(iota_s == fi, cz, nz)
        d = (px - cx) ** 2 + (py - cy) ** 2 + (pz - cz) ** 2
        dist = jnp.minimum(dist, d)
        m = jnp.max(dist)
        far = jnp.min(jnp.where(dist == m, lin, jnp.float32(1e9)))
        return dist, far, nx, ny, nz

    st0 = (jnp.full((32, 128), 1e10, jnp.float32), jnp.float32(0.0),
           jnp.zeros((1, S), jnp.float32), jnp.zeros((1, S), jnp.float32),
           jnp.zeros((1, S), jnp.float32))
    _, _, nx, ny, nz = jax.lax.fori_loop(0, S, body, st0)
    nx_ref[...] = nx
    ny_ref[...] = ny
    nz_ref[...] = nz


def _sa_kernel(pc2d_ref, pcT_ref, featsT_ref, nxyz_ref, *rest):
    prm = rest[:52]
    out_ref = rest[52]
    p_scr = rest[53]
    w_scr = rest[54]

    pc2d = pc2d_ref[...]
    pcT = pcT_ref[...]
    featsT = featsT_ref[...]
    nxyz = nxyz_ref[...]

    nn = jnp.sum(nxyz * nxyz, axis=1, keepdims=True)
    pp = jnp.sum(pcT * pcT, axis=0, keepdims=True)
    cross = jnp.dot(nxyz, pcT, preferred_element_type=jnp.float32)
    d = nn + pp - 2.0 * cross

    for b in range(4):
        (w0f_t, w0x_t, b0, g0, be0,
         w1t, b1, g1, be1,
         w2t, b2, g2, be2) = [r[...] for r in prm[13 * b:13 * (b + 1)]]
        ns = float(NSAMPLES[b])
        m_count = float(S) * ns
        r2 = RADII[b] * RADII[b]

        mask = (d <= r2).astype(jnp.float32)
        c = mask
        sh = 1
        while sh < N:
            c = c + jnp.concatenate(
                [jnp.zeros((S, sh), jnp.float32), c[:, :N - sh]], axis=1)
            sh *= 2
        rank_excl = c - mask
        k_in = jnp.sum(mask, axis=1, keepdims=True)
        k_cl = jnp.minimum(k_in, ns)
        sel = mask * (rank_excl < ns).astype(jnp.float32)
        first = mask * (rank_excl == 0.0).astype(jnp.float32)
        w = sel + first * (ns - k_cl)
        w_scr[...] = w

        p = (jnp.dot(featsT, w0f_t, preferred_element_type=jnp.float32)
             + jnp.dot(pc2d, w0x_t, preferred_element_type=jnp.float32)
             + b0)
        p_scr[...] = p
        q = jnp.dot(nxyz, w0x_t, preferred_element_type=jnp.float32)

        cw = jnp.sum(w, axis=0, keepdims=True)
        sq_sum = jnp.sum(q, axis=0, keepdims=True)
        s1 = jnp.dot(cw, p, preferred_element_type=jnp.float32) - ns * sq_sum
        t_sw = jnp.dot(w, p, preferred_element_type=jnp.float32)
        s1sq = (jnp.dot(cw, p * p, preferred_element_type=jnp.float32)
                - 2.0 * jnp.sum(q * t_sw, axis=0, keepdims=True)
                + ns * jnp.sum(q * q, axis=0, keepdims=True))
        mu1 = s1 / m_count
        var1 = s1sq / m_count - mu1 * mu1
        sc1 = g0 / jnp.sqrt(var1 + 1e-5)
        of1 = be0 - mu1 * sc1

        p_scr[...] = p * sc1 + of1
        qf = q * sc1
        bf16 = jnp.bfloat16

        def h1_tile(t):
            pt = p_scr[pl.ds(t * JT, JT), :]
            return jnp.maximum(pt[None, :, :] - qf[:, None, :], 0.0)

        def w_tile(t):
            return w_scr[:, pl.ds(t * JT, JT)][:, :, None]

        def body_b(t, acc):
            h1 = h1_tile(t)
            wh1 = h1 * w_tile(t)
            g_acc, s_acc = acc
            g_acc = g_acc + jax.lax.dot_general(
                h1.astype(bf16).reshape(S * JT, 128),
                wh1.astype(bf16).reshape(S * JT, 128),
                (((0,), (0,)), ((), ())),
                preferred_element_type=jnp.float32)
            return g_acc, s_acc + jnp.sum(wh1, axis=(0, 1), keepdims=True)

        gram1, ws1 = jax.lax.fori_loop(
            0, NJT, body_b,
            (jnp.zeros((128, 128), jnp.float32), jnp.zeros((1, 1, 128), jnp.float32)))
        ws1 = ws1[0]
        t1 = jnp.dot(ws1, w1t, preferred_element_type=jnp.float32)
        s2 = t1 + m_count * b1
        a1 = jnp.dot(gram1, w1t, preferred_element_type=jnp.float32)
        s2sq = (jnp.sum(w1t * a1, axis=0, keepdims=True)
                + 2.0 * b1 * t1 + m_count * b1 * b1)
        mu2 = s2 / m_count
        var2 = s2sq / m_count - mu2 * mu2
        sc2 = g1 / jnp.sqrt(var2 + 1e-5)
        of2 = be1 - mu2 * sc2
        w2p = (w1t * sc2).astype(bf16)
        b2p = b1 * sc2 + of2

        def h2_tile(t):
            h1 = h1_tile(t).astype(bf16).reshape(S * JT, 128)
            y2 = jnp.dot(h1, w2p, preferred_element_type=jnp.float32)
            return jnp.maximum(y2.reshape(S, JT, 128) + b2p, 0.0)

        def body_c(t, acc):
            h2 = h2_tile(t)
            wh2 = h2 * w_tile(t)
            g_acc, s_acc = acc
            g_acc = g_acc + jax.lax.dot_general(
                h2.astype(bf16).reshape(S * JT, 128),
                wh2.astype(bf16).reshape(S * JT, 128),
                (((0,), (0,)), ((), ())),
                preferred_element_type=jnp.float32)
            return g_acc, s_acc + jnp.sum(wh2, axis=(0, 1), keepdims=True)

        gram2, ws2 = jax.lax.fori_loop(
            0, NJT, body_c,
            (jnp.zeros((128, 128), jnp.float32), jnp.zeros((1, 1, 128), jnp.float32)))
        ws2 = ws2[0]
        t2 = jnp.dot(ws2, w2t, preferred_element_type=jnp.float32)
        s3 = t2 + m_count * b2
        a2 = jnp.dot(gram2, w2t, preferred_element_type=jnp.float32)
        s3sq = (jnp.sum(w2t * a2, axis=0, keepdims=True)
                + 2.0 * b2 * t2 + m_count * b2 * b2)
        mu3 = s3 / m_count
        var3 = s3sq / m_count - mu3 * mu3
        sc3 = g2 / jnp.sqrt(var3 + 1e-5)
        of3 = be2 - mu3 * sc3
        w3p = (w2t * sc3).astype(bf16)
        b3p = b2 * sc3 + of3

        def body_d(t, acc):
            h2 = h2_tile(t).astype(bf16).reshape(S * JT, 128)
            y3 = jnp.dot(h2, w3p, preferred_element_type=jnp.float32)
            h3 = jnp.maximum(y3.reshape(S, JT, 256) + b3p, 0.0)
            hm = jnp.where(w_tile(t) > 0.0, h3, jnp.float32(-1e30))
            return jnp.maximum(acc, jnp.max(hm, axis=1))

        mx0 = jnp.full((S, 256), -1e30, jnp.float32)
        mx = jax.lax.fori_loop(0, NJT, body_d, mx0)
        out_ref[:, 256 * b:256 * (b + 1)] = mx


def _fp_kernel(pc2d_ref, nxyzT_ref, featsT_ref, l1_ref,
               w0i_t_ref, w0f_t_ref, fb0_ref, fg0_ref, fbe0_ref,
               w1t_ref, fb1_ref, fg1_ref, fbe1_ref,
               c1t_ref, c1b_ref, bn1g_ref, bn1be_ref,
               c2w_ref, c2b_ref, out_ref):
    pc2d = pc2d_ref[...]
    nxyzT = nxyzT_ref[...]
    featsT = featsT_ref[...]
    l1 = l1_ref[...]

    pp = jnp.sum(pc2d * pc2d, axis=1, keepdims=True)
    nn = jnp.sum(nxyzT * nxyzT, axis=0, keepdims=True)
    cross = jnp.dot(pc2d, nxyzT, preferred_element_type=jnp.float32)
    d = pp + nn - 2.0 * cross

    colidx = jax.lax.broadcasted_iota(jnp.int32, (N, S), 1).astype(jnp.float32)
    cur = d
    ohs, ws = [], []
    for _ in range(3):
        m = jnp.min(cur, axis=1, keepdims=True)
        cand = jnp.where(cur == m, colidx, jnp.float32(1e9))
        mi = jnp.min(cand, axis=1, keepdims=True)
        oh = (colidx == mi).astype(jnp.float32)
        ohs.append(oh)
        ws.append(1.0 / jnp.maximum(m, 1e-10))
        cur = jnp.where(oh > 0.0, jnp.float32(1e30), cur)
    wsum = ws[0] + ws[1] + ws[2]
    mat = (ohs[0] * (ws[0] / wsum) + ohs[1] * (ws[1] / wsum)
           + ohs[2] * (ws[2] / wsum))

    interp = jnp.dot(mat, l1, preferred_element_type=jnp.float32)

    x = (jnp.dot(interp, w0i_t_ref[...], preferred_element_type=jnp.float32)
         + jnp.dot(featsT, w0f_t_ref[...], preferred_element_type=jnp.float32)
         + fb0_ref[...])
    mu = jnp.mean(x, axis=0, keepdims=True)
    var = jnp.mean(x * x, axis=0, keepdims=True) - mu * mu
    x = jnp.maximum((x - mu) / jnp.sqrt(var + 1e-5) * fg0_ref[...]
                    + fbe0_ref[...], 0.0)

    x = jnp.dot(x, w1t_ref[...], preferred_element_type=jnp.float32) + fb1_ref[...]
    mu = jnp.mean(x, axis=0, keepdims=True)
    var = jnp.mean(x * x, axis=0, keepdims=True) - mu * mu
    x = jnp.maximum((x - mu) / jnp.sqrt(var + 1e-5) * fg1_ref[...]
                    + fbe1_ref[...], 0.0)

    x = jnp.dot(x, c1t_ref[...], preferred_element_type=jnp.float32) + c1b_ref[...]
    mu = jnp.mean(x, axis=0, keepdims=True)
    var = jnp.mean(x * x, axis=0, keepdims=True) - mu * mu
    x = (x - mu) / jnp.sqrt(var + 1e-5) * bn1g_ref[...] + bn1be_ref[...]
    x = jnp.where(x > 0.0, x, 0.01 * x)

    t = jnp.sum(x * c2w_ref[...], axis=1, keepdims=True) + c2b_ref[...]
    tm = jnp.max(t, axis=1, keepdims=True)
    shf = t - tm
    lse = jnp.log(jnp.sum(jnp.exp(shf), axis=1, keepdims=True))
    ls = shf - lse
    out_ref[...] = 1.0 / (1.0 + jnp.exp(-ls))


def kernel(pc, features, params):
    f32 = jnp.float32
    pc2d = pc[0]
    featsT = jnp.transpose(features[0])
    px = pc2d[:, 0].reshape(32, 128)
    py = pc2d[:, 1].reshape(32, 128)
    pz = pc2d[:, 2].reshape(32, 128)

    nx, ny, nz = pl.pallas_call(
        _fps_kernel,
        out_shape=[jax.ShapeDtypeStruct((1, S), f32)] * 3,
    )(px, py, pz)
    nxyz = jnp.concatenate(
        [nx.reshape(S, 1), ny.reshape(S, 1), nz.reshape(S, 1)], axis=1)

    sa_args = [pc2d, jnp.transpose(pc2d), featsT, nxyz]
    for b in range(4):
        w0 = params['sa%d_W0' % b]
        sa_args += [
            jnp.transpose(w0[:, :C_IN]), jnp.transpose(w0[:, C_IN:]),
            params['sa%d_b0' % b].reshape(1, 128),
            params['sa%d_g0' % b].reshape(1, 128),
            params['sa%d_be0' % b].reshape(1, 128),
            jnp.transpose(params['sa%d_W1' % b]),
            params['sa%d_b1' % b].reshape(1, 128),
            params['sa%d_g1' % b].reshape(1, 128),
            params['sa%d_be1' % b].reshape(1, 128),
            jnp.transpose(params['sa%d_W2' % b]),
            params['sa%d_b2' % b].reshape(1, 256),
            params['sa%d_g2' % b].reshape(1, 256),
            params['sa%d_be2' % b].reshape(1, 256),
        ]
    l1 = pl.pallas_call(
        _sa_kernel,
        out_shape=jax.ShapeDtypeStruct((S, 1024), f32),
        scratch_shapes=[pltpu.VMEM((N, 128), f32), pltpu.VMEM((S, N), f32)],
    )(*sa_args)

    fw0 = params['fp_W0']
    fp_args = [
        pc2d, jnp.transpose(nxyz), featsT, l1,
        jnp.transpose(fw0[:, :1024]), jnp.transpose(fw0[:, 1024:]),
        params['fp_b0'].reshape(1, 128),
        params['fp_g0'].reshape(1, 128),
        params['fp_be0'].reshape(1, 128),
        jnp.transpose(params['fp_W1']),
        params['fp_b1'].reshape(1, 64),
        params['fp_g1'].reshape(1, 64),
        params['fp_be1'].reshape(1, 64),
        jnp.transpose(params['c1_W']),
        params['c1_b'].reshape(1, 64),
        params['bn1_g'].reshape(1, 64),
        params['bn1_be'].reshape(1, 64),
        params['c2_W'].reshape(1, 64),
        params['c2_b'].reshape(1, 1),
    ]
    out = pl.pallas_call(
        _fp_kernel,
        out_shape=jax.ShapeDtypeStruct((N, 1), f32),
    )(*fp_args)
    return out.reshape(1, 1, N)


# speedup vs baseline: 1.3658x; 1.0597x over previous
import jax
import jax.numpy as jnp
from jax.experimental import pallas as pl
from jax.experimental.pallas import tpu as pltpu

N = 4096
S = 64
C_IN = 6
RADII = (0.1, 0.4, 0.8, 1.6)
NSAMPLES = (4096, 1024, 256, 64)
JT = 256
NJT = N // JT


def _fps_kernel(px_ref, py_ref, pz_ref, nx_ref, ny_ref, nz_ref):
    px = px_ref[...]
    py = py_ref[...]
    pz = pz_ref[...]
    lin = (jax.lax.broadcasted_iota(jnp.int32, (32, 128), 0) * 128
           + jax.lax.broadcasted_iota(jnp.int32, (32, 128), 1)
           ).astype(jnp.float32)
    iota_s = jax.lax.broadcasted_iota(jnp.int32, (1, S), 1).astype(jnp.float32)

    def body(i, st):
        dist, far, nx, ny, nz = st
        sel = (lin == far).astype(jnp.float32)
        cx = jnp.sum(sel * px)
        cy = jnp.sum(sel * py)
        cz = jnp.sum(sel * pz)
        fi = i.astype(jnp.float32)
        nx = jnp.where(iota_s == fi, cx, nx)
        ny = jnp.where(iota_s == fi, cy, ny)
        nz = jnp.where(iota_s == fi, cz, nz)
        d = (px - cx) ** 2 + (py - cy) ** 2 + (pz - cz) ** 2
        dist = jnp.minimum(dist, d)
        m = jnp.max(dist)
        far = jnp.min(jnp.where(dist == m, lin, jnp.float32(1e9)))
        return dist, far, nx, ny, nz

    st0 = (jnp.full((32, 128), 1e10, jnp.float32), jnp.float32(0.0),
           jnp.zeros((1, S), jnp.float32), jnp.zeros((1, S), jnp.float32),
           jnp.zeros((1, S), jnp.float32))
    _, _, nx, ny, nz = jax.lax.fori_loop(0, S, body, st0)
    nx_ref[...] = nx
    ny_ref[...] = ny
    nz_ref[...] = nz


def _sa_kernel(pc2d_ref, pcT_ref, featsT_ref, nxyz_ref, *rest):
    prm = rest[:52]
    out_ref = rest[52]
    p_scr = rest[53]
    w_scr = rest[54]

    pc2d = pc2d_ref[...]
    pcT = pcT_ref[...]
    featsT = featsT_ref[...]
    nxyz = nxyz_ref[...]

    nn = jnp.sum(nxyz * nxyz, axis=1, keepdims=True)
    pp = jnp.sum(pcT * pcT, axis=0, keepdims=True)
    cross = jnp.dot(nxyz, pcT, preferred_element_type=jnp.float32)
    d = nn + pp - 2.0 * cross

    for b in range(4):
        (w0f_t, w0x_t, b0, g0, be0,
         w1t, b1, g1, be1,
         w2t, b2, g2, be2) = [r[...] for r in prm[13 * b:13 * (b + 1)]]
        ns = float(NSAMPLES[b])
        m_count = float(S) * ns
        r2 = RADII[b] * RADII[b]

        mask = (d <= r2).astype(jnp.float32)
        c = mask
        sh = 1
        while sh < N:
            c = c + jnp.concatenate(
                [jnp.zeros((S, sh), jnp.float32), c[:, :N - sh]], axis=1)
            sh *= 2
        rank_excl = c - mask
        k_in = jnp.sum(mask, axis=1, keepdims=True)
        k_cl = jnp.minimum(k_in, ns)
        sel = mask * (rank_excl < ns).astype(jnp.float32)
        first = mask * (rank_excl == 0.0).astype(jnp.float32)
        w = sel + first * (ns - k_cl)
        w_scr[...] = w

        p = (jnp.dot(featsT, w0f_t, preferred_element_type=jnp.float32)
             + jnp.dot(pc2d, w0x_t, preferred_element_type=jnp.float32)
             + b0)
        p_scr[...] = p
        q = jnp.dot(nxyz, w0x_t, preferred_element_type=jnp.float32)

        cw = jnp.sum(w, axis=0, keepdims=True)
        sq_sum = jnp.sum(q, axis=0, keepdims=True)
        s1 = jnp.dot(cw, p, preferred_element_type=jnp.float32) - ns * sq_sum
        t_sw = jnp.dot(w, p, preferred_element_type=jnp.float32)
        s1sq = (jnp.dot(cw, p * p, preferred_element_type=jnp.float32)
                - 2.0 * jnp.sum(q * t_sw, axis=0, keepdims=True)
                + ns * jnp.sum(q * q, axis=0, keepdims=True))
        mu1 = s1 / m_count
        var1 = s1sq / m_count - mu1 * mu1
        sc1 = g0 / jnp.sqrt(var1 + 1e-5)
        of1 = be0 - mu1 * sc1

        p_scr[...] = p * sc1 + of1
        qf = q * sc1
        bf16 = jnp.bfloat16

        def h1_tile(t):
            pt = p_scr[pl.ds(t * JT, JT), :]
            return jnp.maximum(pt[None, :, :] - qf[:, None, :], 0.0)

        def w_tile(t):
            return w_scr[:, pl.ds(t * JT, JT)][:, :, None]

        def body_b(t, acc):
            h1 = h1_tile(t)
            wh1 = h1 * w_tile(t)
            g_acc, s_acc = acc
            g_acc = g_acc + jax.lax.dot_general(
                h1.astype(bf16).reshape(S * JT, 128),
                wh1.astype(bf16).reshape(S * JT, 128),
                (((0,), (0,)), ((), ())),
                preferred_element_type=jnp.float32)
            return g_acc, s_acc + jnp.sum(wh1, axis=(0, 1), keepdims=True)

        gram1, ws1 = jax.lax.fori_loop(
            0, NJT, body_b,
            (jnp.zeros((128, 128), jnp.float32), jnp.zeros((1, 1, 128), jnp.float32)))
        ws1 = ws1[0]
        t1 = jnp.dot(ws1, w1t, preferred_element_type=jnp.float32)
        s2 = t1 + m_count * b1
        a1 = jnp.dot(gram1, w1t, preferred_element_type=jnp.float32)
        s2sq = (jnp.sum(w1t * a1, axis=0, keepdims=True)
                + 2.0 * b1 * t1 + m_count * b1 * b1)
        mu2 = s2 / m_count
        var2 = s2sq / m_count - mu2 * mu2
        sc2 = g1 / jnp.sqrt(var2 + 1e-5)
        of2 = be1 - mu2 * sc2
        w2p = (w1t * sc2).astype(bf16)
        b2p = b1 * sc2 + of2

        def h2_tile(t):
            h1 = h1_tile(t).astype(bf16).reshape(S * JT, 128)
            y2 = jnp.dot(h1, w2p, preferred_element_type=jnp.float32)
            return jnp.maximum(y2.reshape(S, JT, 128) + b2p, 0.0)

        def body_c(t, acc):
            h2 = h2_tile(t)
            wh2 = h2 * w_tile(t)
            g_acc, s_acc = acc
            g_acc = g_acc + jax.lax.dot_general(
                h2.astype(bf16).reshape(S * JT, 128),
                wh2.astype(bf16).reshape(S * JT, 128),
                (((0,), (0,)), ((), ())),
                preferred_element_type=jnp.float32)
            return g_acc, s_acc + jnp.sum(wh2, axis=(0, 1), keepdims=True)

        gram2, ws2 = jax.lax.fori_loop(
            0, NJT, body_c,
            (jnp.zeros((128, 128), jnp.float32), jnp.zeros((1, 1, 128), jnp.float32)))
        ws2 = ws2[0]
        t2 = jnp.dot(ws2, w2t, preferred_element_type=jnp.float32)
        s3 = t2 + m_count * b2
        a2 = jnp.dot(gram2, w2t, preferred_element_type=jnp.float32)
        s3sq = (jnp.sum(w2t * a2, axis=0, keepdims=True)
                + 2.0 * b2 * t2 + m_count * b2 * b2)
        mu3 = s3 / m_count
        var3 = s3sq / m_count - mu3 * mu3
        sc3 = g2 / jnp.sqrt(var3 + 1e-5)
        of3 = be2 - mu3 * sc3
        w3p = (w2t * sc3).astype(bf16)
        b3p = b2 * sc3 + of3

        def body_d(t, acc):
            h2 = h2_tile(t).astype(bf16).reshape(S * JT, 128)
            y3 = jnp.dot(h2, w3p, preferred_element_type=jnp.float32)
            h3 = jnp.maximum(y3.reshape(S, JT, 256) + b3p, 0.0)
            hm = jnp.where(w_tile(t) > 0.0, h3, jnp.float32(-1e30))
            return jnp.maximum(acc, jnp.max(hm, axis=1))

        mx0 = jnp.full((S, 256), -1e30, jnp.float32)
        mx = jax.lax.fori_loop(0, NJT, body_d, mx0)
        out_ref[:, 256 * b:256 * (b + 1)] = mx


def _fp_kernel(pc2d_ref, nxyzT_ref, featsT_ref, l1_ref,
               w0i_t_ref, w0f_t_ref, fb0_ref, fg0_ref, fbe0_ref,
               w1t_ref, fb1_ref, fg1_ref, fbe1_ref,
               c1t_ref, c1b_ref, bn1g_ref, bn1be_ref,
               c2w_ref, c2b_ref, out_ref):
    pc2d = pc2d_ref[...]
    nxyzT = nxyzT_ref[...]
    featsT = featsT_ref[...]
    l1 = l1_ref[...]

    pp = jnp.sum(pc2d * pc2d, axis=1, keepdims=True)
    nn = jnp.sum(nxyzT * nxyzT, axis=0, keepdims=True)
    cross = jnp.dot(pc2d, nxyzT, preferred_element_type=jnp.float32)
    d = pp + nn - 2.0 * cross

    colidx = jax.lax.broadcasted_iota(jnp.int32, (N, S), 1).astype(jnp.float32)
    cur = d
    ohs, ws = [], []
    for _ in range(3):
        m = jnp.min(cur, axis=1, keepdims=True)
        cand = jnp.where(cur == m, colidx, jnp.float32(1e9))
        mi = jnp.min(cand, axis=1, keepdims=True)
        oh = (colidx == mi).astype(jnp.float32)
        ohs.append(oh)
        ws.append(1.0 / jnp.maximum(m, 1e-10))
        cur = jnp.where(oh > 0.0, jnp.float32(1e30), cur)
    wsum = ws[0] + ws[1] + ws[2]
    mat = (ohs[0] * (ws[0] / wsum) + ohs[1] * (ws[1] / wsum)
           + ohs[2] * (ws[2] / wsum))

    interp = jnp.dot(mat, l1, preferred_element_type=jnp.float32)

    x = (jnp.dot(interp, w0i_t_ref[...], preferred_element_type=jnp.float32)
         + jnp.dot(featsT, w0f_t_ref[...], preferred_element_type=jnp.float32)
         + fb0_ref[...])
    mu = jnp.mean(x, axis=0, keepdims=True)
    var = jnp.mean(x * x, axis=0, keepdims=True) - mu * mu
    x = jnp.maximum((x - mu) / jnp.sqrt(var + 1e-5) * fg0_ref[...]
                    + fbe0_ref[...], 0.0)

    x = jnp.dot(x, w1t_ref[...], preferred_element_type=jnp.float32) + fb1_ref[...]
    mu = jnp.mean(x, axis=0, keepdims=True)
    var = jnp.mean(x * x, axis=0, keepdims=True) - mu * mu
    x = jnp.maximum((x - mu) / jnp.sqrt(var + 1e-5) * fg1_ref[...]
                    + fbe1_ref[...], 0.0)

    x = jnp.dot(x, c1t_ref[...], preferred_element_type=jnp.float32) + c1b_ref[...]
    mu = jnp.mean(x, axis=0, keepdims=True)
    var = jnp.mean(x * x, axis=0, keepdims=True) - mu * mu
    x = (x - mu) / jnp.sqrt(var + 1e-5) * bn1g_ref[...] + bn1be_ref[...]
    x = jnp.where(x > 0.0, x, 0.01 * x)

    t = jnp.sum(x * c2w_ref[...], axis=1, keepdims=True) + c2b_ref[...]
    tm = jnp.max(t, axis=1, keepdims=True)
    shf = t - tm
    lse = jnp.log(jnp.sum(jnp.exp(shf), axis=1, keepdims=True))
    ls = shf - lse
    out_ref[...] = 1.0 / (1.0 + jnp.exp(-ls))


def kernel(pc, features, params):
    f32 = jnp.float32
    pc2d = pc[0]
    featsT = jnp.transpose(features[0])
    px = pc2d[:, 0].reshape(32, 128)
    py = pc2d[:, 1].reshape(32, 128)
    pz = pc2d[:, 2].reshape(32, 128)

    nx, ny, nz = pl.pallas_call(
        _fps_kernel,
        out_shape=[jax.ShapeDtypeStruct((1, S), f32)] * 3,
    )(px, py, pz)
    nxyz = jnp.concatenate(
        [nx.reshape(S, 1), ny.reshape(S, 1), nz.reshape(S, 1)], axis=1)

    sa_args = [pc2d, jnp.transpose(pc2d), featsT, nxyz]
    for b in range(4):
        w0 = params['sa%d_W0' % b]
        sa_args += [
            jnp.transpose(w0[:, :C_IN]), jnp.transpose(w0[:, C_IN:]),
            params['sa%d_b0' % b].reshape(1, 128),
            params['sa%d_g0' % b].reshape(1, 128),
            params['sa%d_be0' % b].reshape(1, 128),
            jnp.transpose(params['sa%d_W1' % b]),
            params['sa%d_b1' % b].reshape(1, 128),
            params['sa%d_g1' % b].reshape(1, 128),
            params['sa%d_be1' % b].reshape(1, 128),
            jnp.transpose(params['sa%d_W2' % b]),
            params['sa%d_b2' % b].reshape(1, 256),
            params['sa%d_g2' % b].reshape(1, 256),
            params['sa%d_be2' % b].reshape(1, 256),
        ]
    l1 = pl.pallas_call(
        _sa_kernel,
        out_shape=jax.ShapeDtypeStruct((S, 1024), f32),
        scratch_shapes=[pltpu.VMEM((N, 128), f32), pltpu.VMEM((S, N), f32)],
    )(*sa_args)

    fw0 = params['fp_W0']
    fp_args = [
        pc2d, jnp.transpose(nxyz), featsT, l1,
        jnp.transpose(fw0[:, :1024]), jnp.transpose(fw0[:, 1024:]),
        params['fp_b0'].reshape(1, 128),
        params['fp_g0'].reshape(1, 128),
        params['fp_be0'].reshape(1, 128),
        jnp.transpose(params['fp_W1']),
        params['fp_b1'].reshape(1, 64),
        params['fp_g1'].reshape(1, 64),
        params['fp_be1'].reshape(1, 64),
        jnp.transpose(params['c1_W']),
        params['c1_b'].reshape(1, 64),
        params['bn1_g'].reshape(1, 64),
        params['bn1_be'].reshape(1, 64),
        params['c2_W'].reshape(1, 64),
        params['c2_b'].reshape(1, 1),
    ]
    out = pl.pallas_call(
        _fp_kernel,
        out_shape=jax.ShapeDtypeStruct((N, 1), f32),
    )(*fp_args)
    return out.reshape(1, 1, N)


# speedup vs baseline: 1.3998x; 1.0249x over previous
import jax
import jax.numpy as jnp
from jax.experimental import pallas as pl
from jax.experimental.pallas import tpu as pltpu

N = 4096
S = 64
C_IN = 6
RADII = (0.1, 0.4, 0.8, 1.6)
NSAMPLES = (4096, 1024, 256, 64)
JT = 256
NJT = N // JT


def _fps_kernel(px_ref, py_ref, pz_ref, nx_ref, ny_ref, nz_ref):
    px = px_ref[...]
    py = py_ref[...]
    pz = pz_ref[...]
    lin = (jax.lax.broadcasted_iota(jnp.int32, (32, 128), 0) * 128
           + jax.lax.broadcasted_iota(jnp.int32, (32, 128), 1)
           ).astype(jnp.float32)
    iota_s = jax.lax.broadcasted_iota(jnp.int32, (1, S), 1).astype(jnp.float32)

    def body(i, st):
        dist, far, nx, ny, nz = st
        sel = (lin == far).astype(jnp.float32)
        cx = jnp.sum(sel * px)
        cy = jnp.sum(sel * py)
        cz = jnp.sum(sel * pz)
        fi = i.astype(jnp.float32)
        nx = jnp.where(iota_s == fi, cx, nx)
        ny = jnp.where(iota_s == fi, cy, ny)
        nz = jnp.where(iota_s == fi, cz, nz)
        d = (px - cx) ** 2 + (py - cy) ** 2 + (pz - cz) ** 2
        dist = jnp.minimum(dist, d)
        m = jnp.max(dist)
        far = jnp.min(jnp.where(dist == m, lin, jnp.float32(1e9)))
        return dist, far, nx, ny, nz

    st0 = (jnp.full((32, 128), 1e10, jnp.float32), jnp.float32(0.0),
           jnp.zeros((1, S), jnp.float32), jnp.zeros((1, S), jnp.float32),
           jnp.zeros((1, S), jnp.float32))
    _, _, nx, ny, nz = jax.lax.fori_loop(0, S, body, st0)
    nx_ref[...] = nx
    ny_ref[...] = ny
    nz_ref[...] = nz


def _sa_kernel(pc2d_ref, pcT_ref, featsT_ref, nxyz_ref, *rest):
    prm = rest[:52]
    out_ref = rest[52]
    p_scr = rest[53]
    w_scr = rest[54]

    pc2d = pc2d_ref[...]
    pcT = pcT_ref[...]
    featsT = featsT_ref[...]
    nxyz = nxyz_ref[...]

    nn = jnp.sum(nxyz * nxyz, axis=1, keepdims=True)
    pp = jnp.sum(pcT * pcT, axis=0, keepdims=True)
    cross = jnp.dot(nxyz, pcT, preferred_element_type=jnp.float32)
    d = nn + pp - 2.0 * cross

    for b in range(4):
        (w0f_t, w0x_t, b0, g0, be0,
         w1t, b1, g1, be1,
         w2t, b2, g2, be2) = [r[...] for r in prm[13 * b:13 * (b + 1)]]
        ns = float(NSAMPLES[b])
        m_count = float(S) * ns
        r2 = RADII[b] * RADII[b]

        mask = (d <= r2).astype(jnp.float32)
        c = mask
        sh = 1
        while sh < N:
            c = c + jnp.concatenate(
                [jnp.zeros((S, sh), jnp.float32), c[:, :N - sh]], axis=1)
            sh *= 2
        rank_excl = c - mask
        k_in = jnp.sum(mask, axis=1, keepdims=True)
        k_cl = jnp.minimum(k_in, ns)
        sel = mask * (rank_excl < ns).astype(jnp.float32)
        first = mask * (rank_excl == 0.0).astype(jnp.float32)
        w = sel + first * (ns - k_cl)
        w_scr[...] = w

        p = (jnp.dot(featsT, w0f_t, preferred_element_type=jnp.float32)
             + jnp.dot(pc2d, w0x_t, preferred_element_type=jnp.float32)
             + b0)
        p_scr[...] = p
        q = jnp.dot(nxyz, w0x_t, preferred_element_type=jnp.float32)

        cw = jnp.sum(w, axis=0, keepdims=True)
        sq_sum = jnp.sum(q, axis=0, keepdims=True)
        s1 = jnp.dot(cw, p, preferred_element_type=jnp.float32) - ns * sq_sum
        t_sw = jnp.dot(w, p, preferred_element_type=jnp.float32)
        s1sq = (jnp.dot(cw, p * p, preferred_element_type=jnp.float32)
                - 2.0 * jnp.sum(q * t_sw, axis=0, keepdims=True)
                + ns * jnp.sum(q * q, axis=0, keepdims=True))
        mu1 = s1 / m_count
        var1 = s1sq / m_count - mu1 * mu1
        sc1 = g0 / jnp.sqrt(var1 + 1e-5)
        of1 = be0 - mu1 * sc1

        p_scr[...] = p * sc1 + of1
        qf = q * sc1
        bf16 = jnp.bfloat16

        def h1_tile(t):
            pt = p_scr[pl.ds(t * JT, JT), :]
            return jnp.maximum(pt[None, :, :] - qf[:, None, :], 0.0)

        def w_tile(t):
            return w_scr[:, pl.ds(t * JT, JT)][:, :, None]

        w1tb = w1t.astype(bf16)

        def body_b(t, acc):
            h1 = h1_tile(t).astype(bf16).reshape(S * JT, 128)
            y2 = (jnp.dot(h1, w1tb, preferred_element_type=jnp.float32) + b1
                  ).reshape(S, JT, 128)
            wy2 = y2 * w_tile(t)
            sa, sb = acc
            return (sa + jnp.sum(wy2, axis=(0, 1), keepdims=True),
                    sb + jnp.sum(wy2 * y2, axis=(0, 1), keepdims=True))

        z128 = jnp.zeros((1, 1, 128), jnp.float32)
        s2, s2sq = jax.lax.fori_loop(0, NJT, body_b, (z128, z128))
        s2, s2sq = s2[0], s2sq[0]
        mu2 = s2 / m_count
        var2 = s2sq / m_count - mu2 * mu2
        sc2 = g1 / jnp.sqrt(var2 + 1e-5)
        of2 = be1 - mu2 * sc2
        w2p = (w1t * sc2).astype(bf16)
        b2p = b1 * sc2 + of2

        def h2_tile(t):
            h1 = h1_tile(t).astype(bf16).reshape(S * JT, 128)
            y2 = jnp.dot(h1, w2p, preferred_element_type=jnp.float32)
            return jnp.maximum(y2.reshape(S, JT, 128) + b2p, 0.0)

        w2tb = w2t.astype(bf16)

        def body_c(t, acc):
            h2 = h2_tile(t).astype(bf16).reshape(S * JT, 128)
            y3 = (jnp.dot(h2, w2tb, preferred_element_type=jnp.float32) + b2
                  ).reshape(S, JT, 256)
            wy3 = y3 * w_tile(t)
            sa, sb = acc
            return (sa + jnp.sum(wy3, axis=(0, 1), keepdims=True),
                    sb + jnp.sum(wy3 * y3, axis=(0, 1), keepdims=True))

        z256 = jnp.zeros((1, 1, 256), jnp.float32)
        s3, s3sq = jax.lax.fori_loop(0, NJT, body_c, (z256, z256))
        s3, s3sq = s3[0], s3sq[0]
        mu3 = s3 / m_count
        var3 = s3sq / m_count - mu3 * mu3
        sc3 = g2 / jnp.sqrt(var3 + 1e-5)
        of3 = be2 - mu3 * sc3
        w3p = (w2t * sc3).astype(bf16)
        b3p = b2 * sc3 + of3

        def body_d(t, acc):
            h2 = h2_tile(t).astype(bf16).reshape(S * JT, 128)
            y3 = jnp.dot(h2, w3p, preferred_element_type=jnp.float32)
            h3 = jnp.maximum(y3.reshape(S, JT, 256) + b3p, 0.0)
            hm = jnp.where(w_tile(t) > 0.0, h3, jnp.float32(-1e30))
            return jnp.maximum(acc, jnp.max(hm, axis=1))

        mx0 = jnp.full((S, 256), -1e30, jnp.float32)
        mx = jax.lax.fori_loop(0, NJT, body_d, mx0)
        out_ref[:, 256 * b:256 * (b + 1)] = mx


def _fp_kernel(pc2d_ref, nxyzT_ref, featsT_ref, l1_ref,
               w0i_t_ref, w0f_t_ref, fb0_ref, fg0_ref, fbe0_ref,
               w1t_ref, fb1_ref, fg1_ref, fbe1_ref,
               c1t_ref, c1b_ref, bn1g_ref, bn1be_ref,
               c2w_ref, c2b_ref, out_ref):
    pc2d = pc2d_ref[...]
    nxyzT = nxyzT_ref[...]
    featsT = featsT_ref[...]
    l1 = l1_ref[...]

    pp = jnp.sum(pc2d * pc2d, axis=1, keepdims=True)
    nn = jnp.sum(nxyzT * nxyzT, axis=0, keepdims=True)
    cross = jnp.dot(pc2d, nxyzT, preferred_element_type=jnp.float32)
    d = pp + nn - 2.0 * cross

    colidx = jax.lax.broadcasted_iota(jnp.int32, (N, S), 1).astype(jnp.float32)
    cur = d
    ohs, ws = [], []
    for _ in range(3):
        m = jnp.min(cur, axis=1, keepdims=True)
        cand = jnp.where(cur == m, colidx, jnp.float32(1e9))
        mi = jnp.min(cand, axis=1, keepdims=True)
        oh = (colidx == mi).astype(jnp.float32)
        ohs.append(oh)
        ws.append(1.0 / jnp.maximum(m, 1e-10))
        cur = jnp.where(oh > 0.0, jnp.float32(1e30), cur)
    wsum = ws[0] + ws[1] + ws[2]
    mat = (ohs[0] * (ws[0] / wsum) + ohs[1] * (ws[1] / wsum)
           + ohs[2] * (ws[2] / wsum))

    interp = jnp.dot(mat, l1, preferred_element_type=jnp.float32)

    x = (jnp.dot(interp, w0i_t_ref[...], preferred_element_type=jnp.float32)
         + jnp.dot(featsT, w0f_t_ref[...], preferred_element_type=jnp.float32)
         + fb0_ref[...])
    mu = jnp.mean(x, axis=0, keepdims=True)
    var = jnp.mean(x * x, axis=0, keepdims=True) - mu * mu
    x = jnp.maximum((x - mu) / jnp.sqrt(var + 1e-5) * fg0_ref[...]
                    + fbe0_ref[...], 0.0)

    x = jnp.dot(x, w1t_ref[...], preferred_element_type=jnp.float32) + fb1_ref[...]
    mu = jnp.mean(x, axis=0, keepdims=True)
    var = jnp.mean(x * x, axis=0, keepdims=True) - mu * mu
    x = jnp.maximum((x - mu) / jnp.sqrt(var + 1e-5) * fg1_ref[...]
                    + fbe1_ref[...], 0.0)

    x = jnp.dot(x, c1t_ref[...], preferred_element_type=jnp.float32) + c1b_ref[...]
    mu = jnp.mean(x, axis=0, keepdims=True)
    var = jnp.mean(x * x, axis=0, keepdims=True) - mu * mu
    x = (x - mu) / jnp.sqrt(var + 1e-5) * bn1g_ref[...] + bn1be_ref[...]
    x = jnp.where(x > 0.0, x, 0.01 * x)

    t = jnp.sum(x * c2w_ref[...], axis=1, keepdims=True) + c2b_ref[...]
    tm = jnp.max(t, axis=1, keepdims=True)
    shf = t - tm
    lse = jnp.log(jnp.sum(jnp.exp(shf), axis=1, keepdims=True))
    ls = shf - lse
    out_ref[...] = 1.0 / (1.0 + jnp.exp(-ls))


def kernel(pc, features, params):
    f32 = jnp.float32
    pc2d = pc[0]
    featsT = jnp.transpose(features[0])
    px = pc2d[:, 0].reshape(32, 128)
    py = pc2d[:, 1].reshape(32, 128)
    pz = pc2d[:, 2].reshape(32, 128)

    nx, ny, nz = pl.pallas_call(
        _fps_kernel,
        out_shape=[jax.ShapeDtypeStruct((1, S), f32)] * 3,
    )(px, py, pz)
    nxyz = jnp.concatenate(
        [nx.reshape(S, 1), ny.reshape(S, 1), nz.reshape(S, 1)], axis=1)

    sa_args = [pc2d, jnp.transpose(pc2d), featsT, nxyz]
    for b in range(4):
        w0 = params['sa%d_W0' % b]
        sa_args += [
            jnp.transpose(w0[:, :C_IN]), jnp.transpose(w0[:, C_IN:]),
            params['sa%d_b0' % b].reshape(1, 128),
            params['sa%d_g0' % b].reshape(1, 128),
            params['sa%d_be0' % b].reshape(1, 128),
            jnp.transpose(params['sa%d_W1' % b]),
            params['sa%d_b1' % b].reshape(1, 128),
            params['sa%d_g1' % b].reshape(1, 128),
            params['sa%d_be1' % b].reshape(1, 128),
            jnp.transpose(params['sa%d_W2' % b]),
            params['sa%d_b2' % b].reshape(1, 256),
            params['sa%d_g2' % b].reshape(1, 256),
            params['sa%d_be2' % b].reshape(1, 256),
        ]
    l1 = pl.pallas_call(
        _sa_kernel,
        out_shape=jax.ShapeDtypeStruct((S, 1024), f32),
        scratch_shapes=[pltpu.VMEM((N, 128), f32), pltpu.VMEM((S, N), f32)],
    )(*sa_args)

    fw0 = params['fp_W0']
    fp_args = [
        pc2d, jnp.transpose(nxyz), featsT, l1,
        jnp.transpose(fw0[:, :1024]), jnp.transpose(fw0[:, 1024:]),
        params['fp_b0'].reshape(1, 128),
        params['fp_g0'].reshape(1, 128),
        params['fp_be0'].reshape(1, 128),
        jnp.transpose(params['fp_W1']),
        params['fp_b1'].reshape(1, 64),
        params['fp_g1'].reshape(1, 64),
        params['fp_be1'].reshape(1, 64),
        jnp.transpose(params['c1_W']),
        params['c1_b'].reshape(1, 64),
        params['bn1_g'].reshape(1, 64),
        params['bn1_be'].reshape(1, 64),
        params['c2_W'].reshape(1, 64),
        params['c2_b'].reshape(1, 1),
    ]
    out = pl.pallas_call(
        _fp_kernel,
        out_shape=jax.ShapeDtypeStruct((N, 1), f32),
    )(*fp_args)
    return out.reshape(1, 1, N)


# speedup vs baseline: 1.6786x; 1.1991x over previous
import jax
import jax.numpy as jnp
from jax.experimental import pallas as pl
from jax.experimental.pallas import tpu as pltpu

N = 4096
S = 64
C_IN = 6
RADII = (0.1, 0.4, 0.8, 1.6)
NSAMPLES = (4096, 1024, 256, 64)
JT = 256
NJT = N // JT


def _fps_kernel(px_ref, py_ref, pz_ref, nx_ref, ny_ref, nz_ref):
    px = px_ref[...]
    py = py_ref[...]
    pz = pz_ref[...]
    lin = (jax.lax.broadcasted_iota(jnp.int32, (32, 128), 0) * 128
           + jax.lax.broadcasted_iota(jnp.int32, (32, 128), 1)
           ).astype(jnp.float32)
    iota_s = jax.lax.broadcasted_iota(jnp.int32, (1, S), 1).astype(jnp.float32)

    def body(i, st):
        dist, far, nx, ny, nz = st
        sel = (lin == far).astype(jnp.float32)
        cx = jnp.sum(sel * px)
        cy = jnp.sum(sel * py)
        cz = jnp.sum(sel * pz)
        fi = i.astype(jnp.float32)
        nx = jnp.where(iota_s == fi, cx, nx)
        ny = jnp.where(iota_s == fi, cy, ny)
        nz = jnp.where(iota_s == fi, cz, nz)
        d = (px - cx) ** 2 + (py - cy) ** 2 + (pz - cz) ** 2
        dist = jnp.minimum(dist, d)
        m = jnp.max(dist)
        far = jnp.min(jnp.where(dist == m, lin, jnp.float32(1e9)))
        return dist, far, nx, ny, nz

    st0 = (jnp.full((32, 128), 1e10, jnp.float32), jnp.float32(0.0),
           jnp.zeros((1, S), jnp.float32), jnp.zeros((1, S), jnp.float32),
           jnp.zeros((1, S), jnp.float32))
    _, _, nx, ny, nz = jax.lax.fori_loop(0, S, body, st0)
    nx_ref[...] = nx
    ny_ref[...] = ny
    nz_ref[...] = nz


def _sa_kernel(pc2d_ref, pcT_ref, featsT_ref, nxyz_ref, *rest):
    prm = rest[:52]
    out_ref = rest[52]
    p_scr = rest[53]
    w_scr = rest[54]

    pc2d = pc2d_ref[...]
    pcT = pcT_ref[...]
    featsT = featsT_ref[...]
    nxyz = nxyz_ref[...]

    nn = jnp.sum(nxyz * nxyz, axis=1, keepdims=True)
    pp = jnp.sum(pcT * pcT, axis=0, keepdims=True)
    cross = jnp.dot(nxyz, pcT, preferred_element_type=jnp.float32)
    d = nn + pp - 2.0 * cross

    for b in range(4):
        (w0f_t, w0x_t, b0, g0, be0,
         w1t, b1, g1, be1,
         w2t, b2, g2, be2) = [r[...] for r in prm[13 * b:13 * (b + 1)]]
        ns = float(NSAMPLES[b])
        m_count = float(S) * ns
        r2 = RADII[b] * RADII[b]

        mask = (d <= r2).astype(jnp.float32)
        c = mask
        sh = 1
        while sh < N:
            c = c + jnp.concatenate(
                [jnp.zeros((S, sh), jnp.float32), c[:, :N - sh]], axis=1)
            sh *= 2
        rank_excl = c - mask
        k_in = jnp.sum(mask, axis=1, keepdims=True)
        k_cl = jnp.minimum(k_in, ns)
        sel = mask * (rank_excl < ns).astype(jnp.float32)
        first = mask * (rank_excl == 0.0).astype(jnp.float32)
        w = sel + first * (ns - k_cl)
        w_scr[...] = w

        p = (jnp.dot(featsT, w0f_t, preferred_element_type=jnp.float32)
             + jnp.dot(pc2d, w0x_t, preferred_element_type=jnp.float32)
             + b0)
        p_scr[...] = p
        q = jnp.dot(nxyz, w0x_t, preferred_element_type=jnp.float32)

        cw = jnp.sum(w, axis=0, keepdims=True)
        sq_sum = jnp.sum(q, axis=0, keepdims=True)
        s1 = jnp.dot(cw, p, preferred_element_type=jnp.float32) - ns * sq_sum
        t_sw = jnp.dot(w, p, preferred_element_type=jnp.float32)
        s1sq = (jnp.dot(cw, p * p, preferred_element_type=jnp.float32)
                - 2.0 * jnp.sum(q * t_sw, axis=0, keepdims=True)
                + ns * jnp.sum(q * q, axis=0, keepdims=True))
        mu1 = s1 / m_count
        var1 = s1sq / m_count - mu1 * mu1
        sc1 = g0 / jnp.sqrt(var1 + 1e-5)
        of1 = be0 - mu1 * sc1

        p_scr[...] = p * sc1 + of1
        qf = q * sc1
        bf16 = jnp.bfloat16

        def h1_tile(t):
            pt = p_scr[pl.ds(t * JT, JT), :]
            return jnp.maximum(pt[None, :, :] - qf[:, None, :], 0.0)

        def w_tile(t):
            return w_scr[:, pl.ds(t * JT, JT)][:, :, None]

        w1tb = w1t.astype(bf16)

        def body_b(t, acc):
            h1 = h1_tile(t).astype(bf16).reshape(S * JT, 128)
            y2 = (jnp.dot(h1, w1tb, preferred_element_type=jnp.float32) + b1
                  ).reshape(S, JT, 128)
            wy2 = y2 * w_tile(t)
            sa, sb = acc
            return (sa + jnp.sum(wy2, axis=(0, 1), keepdims=True),
                    sb + jnp.sum(wy2 * y2, axis=(0, 1), keepdims=True))

        z128 = jnp.zeros((1, 1, 128), jnp.float32)
        s2, s2sq = jax.lax.fori_loop(0, NJT, body_b, (z128, z128))
        s2, s2sq = s2[0], s2sq[0]
        mu2 = s2 / m_count
        var2 = s2sq / m_count - mu2 * mu2
        sc2 = g1 / jnp.sqrt(var2 + 1e-5)
        of2 = be1 - mu2 * sc2
        w2p = (w1t * sc2).astype(bf16)
        b2p = b1 * sc2 + of2

        def h2_tile(t):
            h1 = h1_tile(t).astype(bf16).reshape(S * JT, 128)
            y2 = jnp.dot(h1, w2p, preferred_element_type=jnp.float32)
            return jnp.maximum(y2.reshape(S, JT, 128) + b2p, 0.0)

        w2tb = w2t.astype(bf16)

        def body_c(t, acc):
            h2 = h2_tile(t).astype(bf16).reshape(S * JT, 128)
            y3 = (jnp.dot(h2, w2tb, preferred_element_type=jnp.float32) + b2
                  ).reshape(S, JT, 256)
            wt = w_tile(t)
            wy3 = y3 * wt
            sa, sb, mx_a, mn_a = acc
            selm = wt > 0.0
            mx_a = jnp.maximum(mx_a, jnp.max(
                jnp.where(selm, y3, jnp.float32(-1e30)), axis=1))
            mn_a = jnp.minimum(mn_a, jnp.min(
                jnp.where(selm, y3, jnp.float32(1e30)), axis=1))
            return (sa + jnp.sum(wy3, axis=(0, 1), keepdims=True),
                    sb + jnp.sum(wy3 * y3, axis=(0, 1), keepdims=True),
                    mx_a, mn_a)

        z256 = jnp.zeros((1, 1, 256), jnp.float32)
        s3, s3sq, mx_raw, mn_raw = jax.lax.fori_loop(
            0, NJT, body_c,
            (z256, z256, jnp.full((S, 256), -1e30, jnp.float32),
             jnp.full((S, 256), 1e30, jnp.float32)))
        s3, s3sq = s3[0], s3sq[0]
        mu3 = s3 / m_count
        var3 = s3sq / m_count - mu3 * mu3
        sc3 = g2 / jnp.sqrt(var3 + 1e-5)
        of3 = be2 - mu3 * sc3
        pick = jnp.where(sc3 > 0.0, mx_raw, mn_raw)
        out_ref[:, 256 * b:256 * (b + 1)] = jnp.maximum(pick * sc3 + of3, 0.0)


def _fp_kernel(pc2d_ref, nxyzT_ref, featsT_ref, l1_ref,
               w0i_t_ref, w0f_t_ref, fb0_ref, fg0_ref, fbe0_ref,
               w1t_ref, fb1_ref, fg1_ref, fbe1_ref,
               c1t_ref, c1b_ref, bn1g_ref, bn1be_ref,
               c2w_ref, c2b_ref, out_ref):
    pc2d = pc2d_ref[...]
    nxyzT = nxyzT_ref[...]
    featsT = featsT_ref[...]
    l1 = l1_ref[...]

    pp = jnp.sum(pc2d * pc2d, axis=1, keepdims=True)
    nn = jnp.sum(nxyzT * nxyzT, axis=0, keepdims=True)
    cross = jnp.dot(pc2d, nxyzT, preferred_element_type=jnp.float32)
    d = pp + nn - 2.0 * cross

    colidx = jax.lax.broadcasted_iota(jnp.int32, (N, S), 1).astype(jnp.float32)
    cur = d
    ohs, ws = [], []
    for _ in range(3):
        m = jnp.min(cur, axis=1, keepdims=True)
        cand = jnp.where(cur == m, colidx, jnp.float32(1e9))
        mi = jnp.min(cand, axis=1, keepdims=True)
        oh = (colidx == mi).astype(jnp.float32)
        ohs.append(oh)
        ws.append(1.0 / jnp.maximum(m, 1e-10))
        cur = jnp.where(oh > 0.0, jnp.float32(1e30), cur)
    wsum = ws[0] + ws[1] + ws[2]
    mat = (ohs[0] * (ws[0] / wsum) + ohs[1] * (ws[1] / wsum)
           + ohs[2] * (ws[2] / wsum))

    interp = jnp.dot(mat, l1, preferred_element_type=jnp.float32)

    x = (jnp.dot(interp, w0i_t_ref[...], preferred_element_type=jnp.float32)
         + jnp.dot(featsT, w0f_t_ref[...], preferred_element_type=jnp.float32)
         + fb0_ref[...])
    mu = jnp.mean(x, axis=0, keepdims=True)
    var = jnp.mean(x * x, axis=0, keepdims=True) - mu * mu
    x = jnp.maximum((x - mu) / jnp.sqrt(var + 1e-5) * fg0_ref[...]
                    + fbe0_ref[...], 0.0)

    x = jnp.dot(x, w1t_ref[...], preferred_element_type=jnp.float32) + fb1_ref[...]
    mu = jnp.mean(x, axis=0, keepdims=True)
    var = jnp.mean(x * x, axis=0, keepdims=True) - mu * mu
    x = jnp.maximum((x - mu) / jnp.sqrt(var + 1e-5) * fg1_ref[...]
                    + fbe1_ref[...], 0.0)

    x = jnp.dot(x, c1t_ref[...], preferred_element_type=jnp.float32) + c1b_ref[...]
    mu = jnp.mean(x, axis=0, keepdims=True)
    var = jnp.mean(x * x, axis=0, keepdims=True) - mu * mu
    x = (x - mu) / jnp.sqrt(var + 1e-5) * bn1g_ref[...] + bn1be_ref[...]
    x = jnp.where(x > 0.0, x, 0.01 * x)

    t = jnp.sum(x * c2w_ref[...], axis=1, keepdims=True) + c2b_ref[...]
    tm = jnp.max(t, axis=1, keepdims=True)
    shf = t - tm
    lse = jnp.log(jnp.sum(jnp.exp(shf), axis=1, keepdims=True))
    ls = shf - lse
    out_ref[...] = 1.0 / (1.0 + jnp.exp(-ls))


def kernel(pc, features, params):
    f32 = jnp.float32
    pc2d = pc[0]
    featsT = jnp.transpose(features[0])
    px = pc2d[:, 0].reshape(32, 128)
    py = pc2d[:, 1].reshape(32, 128)
    pz = pc2d[:, 2].reshape(32, 128)

    nx, ny, nz = pl.pallas_call(
        _fps_kernel,
        out_shape=[jax.ShapeDtypeStruct((1, S), f32)] * 3,
    )(px, py, pz)
    nxyz = jnp.concatenate(
        [nx.reshape(S, 1), ny.reshape(S, 1), nz.reshape(S, 1)], axis=1)

    sa_args = [pc2d, jnp.transpose(pc2d), featsT, nxyz]
    for b in range(4):
        w0 = params['sa%d_W0' % b]
        sa_args += [
            jnp.transpose(w0[:, :C_IN]), jnp.transpose(w0[:, C_IN:]),
            params['sa%d_b0' % b].reshape(1, 128),
            params['sa%d_g0' % b].reshape(1, 128),
            params['sa%d_be0' % b].reshape(1, 128),
            jnp.transpose(params['sa%d_W1' % b]),
            params['sa%d_b1' % b].reshape(1, 128),
            params['sa%d_g1' % b].reshape(1, 128),
            params['sa%d_be1' % b].reshape(1, 128),
            jnp.transpose(params['sa%d_W2' % b]),
            params['sa%d_b2' % b].reshape(1, 256),
            params['sa%d_g2' % b].reshape(1, 256),
            params['sa%d_be2' % b].reshape(1, 256),
        ]
    l1 = pl.pallas_call(
        _sa_kernel,
        out_shape=jax.ShapeDtypeStruct((S, 1024), f32),
        scratch_shapes=[pltpu.VMEM((N, 128), f32), pltpu.VMEM((S, N), f32)],
    )(*sa_args)

    fw0 = params['fp_W0']
    fp_args = [
        pc2d, jnp.transpose(nxyz), featsT, l1,
        jnp.transpose(fw0[:, :1024]), jnp.transpose(fw0[:, 1024:]),
        params['fp_b0'].reshape(1, 128),
        params['fp_g0'].reshape(1, 128),
        params['fp_be0'].reshape(1, 128),
        jnp.transpose(params['fp_W1']),
        params['fp_b1'].reshape(1, 64),
        params['fp_g1'].reshape(1, 64),
        params['fp_be1'].reshape(1, 64),
        jnp.transpose(params['c1_W']),
        params['c1_b'].reshape(1, 64),
        params['bn1_g'].reshape(1, 64),
        params['bn1_be'].reshape(1, 64),
        params['c2_W'].reshape(1, 64),
        params['c2_b'].reshape(1, 1),
    ]
    out = pl.pallas_call(
        _fp_kernel,
        out_shape=jax.ShapeDtypeStruct((N, 1), f32),
    )(*fp_args)
    return out.reshape(1, 1, N)
